# baseline (speedup 1.0000x reference)
import functools

import jax
import jax.numpy as jnp
from jax import lax
from jax.experimental import pallas as pl
from jax.experimental.pallas import tpu as pltpu
from jax.experimental.pallas import tpu_sc as plsc

_B, _N = 64, 32768
_K1 = int(0.01 * (_N - 1)) + 1
_NVEC = _N // 16
_NBKT = 4096
_HIST = _NBKT * 16
_CAP = 32000
_MASK31 = 0x7FFFFFFF


def _sc_select_body(keys_hbm, zeros_hbm, out_hbm, row_v, hist_v, cand_v,
                    ebuf_v, sem):
    wid = lax.axis_index("s") * 2 + lax.axis_index("c")
    lane = lax.broadcasted_iota(jnp.int32, (16,), 0)
    ones = jnp.ones((16,), jnp.int32)

    for r_i in range(2):
        r = wid * 2 + r_i
        pltpu.sync_copy(zeros_hbm, hist_v)
        pltpu.sync_copy(keys_hbm.at[r], row_v)

        def p1(i, carry):
            b = row_v[pl.ds(i * 16, 16)]
            key = b ^ ((b >> 31) & _MASK31)
            row_v[pl.ds(i * 16, 16)] = key
            bucket = (key >> 20) + 2048
            plsc.addupdate_scatter(hist_v, [bucket * 16 + lane], ones)
            return carry

        lax.fori_loop(0, _NVEC, p1, jnp.int32(0))

        def scond(carry):
            _, cum = carry
            return cum < _K1

        def sbody(carry):
            b, cum = carry
            return b - 1, cum + jnp.sum(hist_v[pl.ds(b * 16, 16)])

        bminus, ec = lax.while_loop(scond, sbody,
                                    (jnp.int32(_NBKT - 1), jnp.int32(0)))
        b0 = bminus + 1
        prefix = (b0 - 2048) << 20

        def descend(ref, n):
            nv = (n + 15) // 16

            def outer(j, res):
                cand = res + (jnp.int32(1) << (jnp.int32(19) - j))

                def inner(i, acc):
                    key = ref[pl.ds(i * 16, 16)]
                    valid = (lane + i * 16) < n
                    hit = jnp.where(valid & (key >= cand), 1, 0)
                    return acc + hit.astype(jnp.int32)

                acc = lax.fori_loop(0, nv, inner, jnp.zeros((16,), jnp.int32))
                return jnp.where(jnp.sum(acc) >= _K1, cand, res)

            return lax.fori_loop(0, 20, outer, prefix)

        def fast_path(_):
            def p2(i, off):
                key = row_v[pl.ds(i * 16, 16)]
                msk = key >= prefix
                plsc.store_compressed(cand_v.at[pl.ds(off, 16)], key,
                                      mask=msk)
                return off + jnp.max(plsc.all_reduce_population_count(msk))

            lax.fori_loop(0, _NVEC, p2, jnp.int32(0))
            return descend(cand_v, ec)

        def slow_path(_):
            return descend(row_v, jnp.int32(_N))

        tcode = lax.cond(ec <= _CAP, fast_path, slow_path, jnp.int32(0))
        ebuf_v[pl.ds(0, 16)] = jnp.full((16,), tcode)
        pltpu.sync_copy(ebuf_v, out_hbm.at[r])


def _tc_finish_body(logits_ref, tgt_ref, tcode_ref, out_ref):
    x = logits_ref[...]
    b = lax.bitcast_convert_type(x, jnp.int32)
    key = b ^ ((b >> 31) & _MASK31)
    tcode = tcode_ref[:, 0:1]

    m = jnp.max(x, axis=1, keepdims=True)
    gt = key > tcode
    c = jnp.sum(gt.astype(jnp.int32), axis=1, keepdims=True)
    tsum = jnp.sum(jnp.where(gt, jnp.exp(10.0 * (x - m)), 0.0), axis=1,
                   keepdims=True)

    tb = tcode ^ ((tcode >> 31) & _MASK31)
    t = lax.bitcast_convert_type(tb, jnp.float32)
    cols = lax.broadcasted_iota(jnp.int32, (_B, _N), 1)
    pos = jnp.sum(jnp.where(cols == tgt_ref[...], x, 0.0), axis=1,
                  keepdims=True)

    et = jnp.exp(10.0 * (t - m))
    ep = jnp.exp(10.0 * (pos - m))
    s = tsum + (_K1 - c).astype(jnp.float32) * et + jnp.where(
        pos < t, ep - et, 0.0)
    loss = jnp.log(s) + 10.0 * m - 10.0 * pos
    out_ref[0, 0] = jnp.sum(loss) / _B


@jax.jit
def kernel(logits, targets):
    keys_i32 = lax.bitcast_convert_type(logits, jnp.int32)
    mesh = plsc.VectorSubcoreMesh(core_axis_name="c", subcore_axis_name="s")
    sc = functools.partial(
        pl.kernel,
        mesh=mesh,
        compiler_params=pltpu.CompilerParams(needs_layout_passes=False),
        out_type=jax.ShapeDtypeStruct((_B, 16), jnp.int32),
        scratch_types=[
            pltpu.VMEM((_N,), jnp.int32),
            pltpu.VMEM((_HIST,), jnp.int32),
            pltpu.VMEM((_CAP + 16,), jnp.int32),
            pltpu.VMEM((16,), jnp.int32),
            pltpu.SemaphoreType.DMA,
        ],
    )(_sc_select_body)
    tcodes = sc(keys_i32, jnp.zeros((_HIST,), jnp.int32))

    out = pl.pallas_call(
        _tc_finish_body,
        out_shape=jax.ShapeDtypeStruct((1, 1), jnp.float32),
        in_specs=[
            pl.BlockSpec(memory_space=pltpu.VMEM),
            pl.BlockSpec(memory_space=pltpu.VMEM),
            pl.BlockSpec(memory_space=pltpu.VMEM),
        ],
        out_specs=pl.BlockSpec(memory_space=pltpu.SMEM),
    )(logits, targets.reshape(_B, 1).astype(jnp.int32), tcodes)
    return out[0, 0]


# speedup vs baseline: 1.6774x; 1.6774x over previous
import functools

import jax
import jax.numpy as jnp
from jax import lax
from jax.experimental import pallas as pl
from jax.experimental.pallas import tpu as pltpu
from jax.experimental.pallas import tpu_sc as plsc

_B, _N = 64, 32768
_K1 = int(0.01 * (_N - 1)) + 1
_NVEC = _N // 16
_NBKT = 4096
_HIST = _NBKT * 16
_CAP = 32000
_MASK31 = 0x7FFFFFFF


def _sc_select_body(keys_hbm, zeros_hbm, out_hbm, row_v, hist_v, cand_v,
                    ebuf_v, sem):
    wid = lax.axis_index("s") * 2 + lax.axis_index("c")
    lane = lax.broadcasted_iota(jnp.int32, (16,), 0)
    ones = jnp.ones((16,), jnp.int32)

    for r_i in range(2):
        r = wid * 2 + r_i
        pltpu.sync_copy(zeros_hbm, hist_v)
        pltpu.sync_copy(keys_hbm.at[r], row_v)

        @plsc.parallel_loop(0, _NVEC, unroll=4)
        def _p1(i):
            b = row_v[pl.ds(i * 16, 16)]
            key = b ^ ((b >> 31) & _MASK31)
            row_v[pl.ds(i * 16, 16)] = key
            bucket = (key >> 20) + 2048
            plsc.addupdate_scatter(hist_v, [bucket * 16 + lane], ones)

        def scond(carry):
            _, cum = carry
            return cum < _K1

        def sbody(carry):
            b, cum = carry
            return b - 1, cum + jnp.sum(hist_v[pl.ds(b * 16, 16)])

        bminus, ec = lax.while_loop(scond, sbody,
                                    (jnp.int32(_NBKT - 1), jnp.int32(0)))
        b0 = bminus + 1
        prefix = (b0 - 2048) << 20

        def descend(ref, n):
            nv = (n + 15) // 16

            def outer(j, res):
                cand = res + (jnp.int32(1) << (jnp.int32(19) - j))

                @plsc.parallel_loop(0, nv, unroll=4,
                                    carry=jnp.zeros((16,), jnp.int32))
                def acc(i, a):
                    key = ref[pl.ds(i * 16, 16)]
                    valid = (lane + i * 16) < n
                    hit = jnp.where(valid & (key >= cand), 1, 0)
                    return a + hit.astype(jnp.int32)

                return jnp.where(jnp.sum(acc) >= _K1, cand, res)

            return lax.fori_loop(0, 20, outer, prefix)

        def fast_path(_):
            @plsc.parallel_loop(0, _NVEC, unroll=4, carry=jnp.int32(0))
            def _off(i, off):
                key = row_v[pl.ds(i * 16, 16)]
                msk = key >= prefix
                plsc.store_compressed(cand_v.at[pl.ds(off, 16)], key,
                                      mask=msk)
                return off + plsc.all_reduce_population_count(msk)[0]

            return descend(cand_v, ec)

        def slow_path(_):
            return descend(row_v, jnp.int32(_N))

        tcode = lax.cond(ec <= _CAP, fast_path, slow_path, jnp.int32(0))
        ebuf_v[pl.ds(0, 16)] = jnp.full((16,), tcode)
        pltpu.sync_copy(ebuf_v, out_hbm.at[r])


def _tc_finish_body(logits_ref, tgt_ref, tcode_ref, out_ref):
    x = logits_ref[...]
    b = lax.bitcast_convert_type(x, jnp.int32)
    key = b ^ ((b >> 31) & _MASK31)
    tcode = tcode_ref[:, 0:1]

    m = jnp.max(x, axis=1, keepdims=True)
    gt = key > tcode
    c = jnp.sum(gt.astype(jnp.int32), axis=1, keepdims=True)
    tsum = jnp.sum(jnp.where(gt, jnp.exp(10.0 * (x - m)), 0.0), axis=1,
                   keepdims=True)

    tb = tcode ^ ((tcode >> 31) & _MASK31)
    t = lax.bitcast_convert_type(tb, jnp.float32)
    cols = lax.broadcasted_iota(jnp.int32, (_B, _N), 1)
    pos = jnp.sum(jnp.where(cols == tgt_ref[...], x, 0.0), axis=1,
                  keepdims=True)

    et = jnp.exp(10.0 * (t - m))
    ep = jnp.exp(10.0 * (pos - m))
    s = tsum + (_K1 - c).astype(jnp.float32) * et + jnp.where(
        pos < t, ep - et, 0.0)
    loss = jnp.log(s) + 10.0 * m - 10.0 * pos
    out_ref[0, 0] = jnp.sum(loss) / _B


@jax.jit
def kernel(logits, targets):
    keys_i32 = lax.bitcast_convert_type(logits, jnp.int32)
    mesh = plsc.VectorSubcoreMesh(core_axis_name="c", subcore_axis_name="s")
    sc = functools.partial(
        pl.kernel,
        mesh=mesh,
        compiler_params=pltpu.CompilerParams(needs_layout_passes=False),
        out_type=jax.ShapeDtypeStruct((_B, 16), jnp.int32),
        scratch_types=[
            pltpu.VMEM((_N,), jnp.int32),
            pltpu.VMEM((_HIST,), jnp.int32),
            pltpu.VMEM((_CAP + 16,), jnp.int32),
            pltpu.VMEM((16,), jnp.int32),
            pltpu.SemaphoreType.DMA,
        ],
    )(_sc_select_body)
    tcodes = sc(keys_i32, jnp.zeros((_HIST,), jnp.int32))

    out = pl.pallas_call(
        _tc_finish_body,
        out_shape=jax.ShapeDtypeStruct((1, 1), jnp.float32),
        in_specs=[
            pl.BlockSpec(memory_space=pltpu.VMEM),
            pl.BlockSpec(memory_space=pltpu.VMEM),
            pl.BlockSpec(memory_space=pltpu.VMEM),
        ],
        out_specs=pl.BlockSpec(memory_space=pltpu.SMEM),
    )(logits, targets.reshape(_B, 1).astype(jnp.int32), tcodes)
    return out[0, 0]


# speedup vs baseline: 3.2399x; 1.9316x over previous
import functools

import jax
import jax.numpy as jnp
from jax import lax
from jax.experimental import pallas as pl
from jax.experimental.pallas import tpu as pltpu
from jax.experimental.pallas import tpu_sc as plsc

_B, _N = 64, 32768
_K1 = int(0.01 * (_N - 1)) + 1
_NVEC = _N // 16
_CAP = 32000
_MASK31 = 0x7FFFFFFF
_TAU = 0x40000000


def _sc_select_body(keys_hbm, out_hbm, row0_v, row1_v, cand0_v, cand1_v,
                    ebuf_v, sem):
    wid = lax.axis_index("s") * 2 + lax.axis_index("c")
    lane = lax.broadcasted_iota(jnp.int32, (16,), 0)
    r0 = wid * 2
    pltpu.sync_copy(keys_hbm.at[r0], row0_v)
    pltpu.sync_copy(keys_hbm.at[r0 + 1], row1_v)

    @plsc.parallel_loop(0, _NVEC, unroll=4,
                        carry=(jnp.int32(0), jnp.int32(0)))
    def offs(i, c):
        o0, o1 = c
        b0 = row0_v[pl.ds(i * 16, 16)]
        k0 = b0 ^ ((b0 >> 31) & _MASK31)
        m0 = k0 >= _TAU
        plsc.store_compressed(cand0_v.at[pl.ds(o0, 16)], k0,
                              mask=m0 & (o0 < _CAP))
        b1 = row1_v[pl.ds(i * 16, 16)]
        k1 = b1 ^ ((b1 >> 31) & _MASK31)
        m1 = k1 >= _TAU
        plsc.store_compressed(cand1_v.at[pl.ds(o1, 16)], k1,
                              mask=m1 & (o1 < _CAP))
        return (o0 + plsc.all_reduce_population_count(m0)[0],
                o1 + plsc.all_reduce_population_count(m1)[0])

    pad = jnp.full((16,), -2147483648, jnp.int32)

    for r_i, (row_v, cand_v, off) in enumerate(
            [(row0_v, cand0_v, offs[0]), (row1_v, cand1_v, offs[1])]):

        def fast_path(_, cand_v=cand_v, off=off):
            cand_v[pl.ds(off, 16)] = pad
            nv = (off + 15) // 16

            def outer(j, res):
                cnd = res + (jnp.int32(1) << (jnp.int32(29) - j))

                @plsc.parallel_loop(0, nv, unroll=4,
                                    carry=jnp.zeros((16,), jnp.int32))
                def acc(i, a, cnd=cnd, cand_v=cand_v):
                    key = cand_v[pl.ds(i * 16, 16)]
                    return a + jnp.where(key >= cnd, 1, 0).astype(jnp.int32)

                return jnp.where(jnp.sum(acc) >= _K1, cnd, res)

            return lax.fori_loop(0, 30, outer, jnp.int32(_TAU))

        def slow_path(_, row_v=row_v):
            def outer(j, res):
                cnd = res + (jnp.int32(1) << (jnp.int32(30) - j))

                @plsc.parallel_loop(0, _NVEC, unroll=4,
                                    carry=jnp.zeros((16,), jnp.int32))
                def acc(i, a, cnd=cnd, row_v=row_v):
                    b = row_v[pl.ds(i * 16, 16)]
                    key = b ^ ((b >> 31) & _MASK31)
                    return a + jnp.where(key >= cnd, 1, 0).astype(jnp.int32)

                return jnp.where(jnp.sum(acc) >= _K1, cnd, res)

            return lax.fori_loop(0, 31, outer, jnp.int32(-2147483648))

        ok = (off >= _K1) & (off <= _CAP)
        tcode = lax.cond(ok, fast_path, slow_path, jnp.int32(0))
        ebuf_v[pl.ds(0, 16)] = jnp.full((16,), tcode)
        pltpu.sync_copy(ebuf_v, out_hbm.at[r0 + r_i])


def _tc_finish_body(logits_ref, tgt_ref, tcode_ref, out_ref):
    x = logits_ref[...]
    b = lax.bitcast_convert_type(x, jnp.int32)
    key = b ^ ((b >> 31) & _MASK31)
    tcode = tcode_ref[:, 0:1]

    m = jnp.max(x, axis=1, keepdims=True)
    gt = key > tcode
    c = jnp.sum(gt.astype(jnp.int32), axis=1, keepdims=True)
    tsum = jnp.sum(jnp.where(gt, jnp.exp(10.0 * (x - m)), 0.0), axis=1,
                   keepdims=True)

    tb = tcode ^ ((tcode >> 31) & _MASK31)
    t = lax.bitcast_convert_type(tb, jnp.float32)
    cols = lax.broadcasted_iota(jnp.int32, (_B, _N), 1)
    pos = jnp.sum(jnp.where(cols == tgt_ref[...], x, 0.0), axis=1,
                  keepdims=True)

    et = jnp.exp(10.0 * (t - m))
    ep = jnp.exp(10.0 * (pos - m))
    s = tsum + (_K1 - c).astype(jnp.float32) * et + jnp.where(
        pos < t, ep - et, 0.0)
    loss = jnp.log(s) + 10.0 * m - 10.0 * pos
    out_ref[0, 0] = jnp.sum(loss) / _B


@jax.jit
def kernel(logits, targets):
    keys_i32 = lax.bitcast_convert_type(logits, jnp.int32)
    mesh = plsc.VectorSubcoreMesh(core_axis_name="c", subcore_axis_name="s")
    sc = functools.partial(
        pl.kernel,
        mesh=mesh,
        compiler_params=pltpu.CompilerParams(needs_layout_passes=False),
        out_type=jax.ShapeDtypeStruct((_B, 16), jnp.int32),
        scratch_types=[
            pltpu.VMEM((_N,), jnp.int32),
            pltpu.VMEM((_N,), jnp.int32),
            pltpu.VMEM((_CAP + 16,), jnp.int32),
            pltpu.VMEM((_CAP + 16,), jnp.int32),
            pltpu.VMEM((16,), jnp.int32),
            pltpu.SemaphoreType.DMA,
        ],
    )(_sc_select_body)
    tcodes = sc(keys_i32)

    out = pl.pallas_call(
        _tc_finish_body,
        out_shape=jax.ShapeDtypeStruct((1, 1), jnp.float32),
        in_specs=[
            pl.BlockSpec(memory_space=pltpu.VMEM),
            pl.BlockSpec(memory_space=pltpu.VMEM),
            pl.BlockSpec(memory_space=pltpu.VMEM),
        ],
        out_specs=pl.BlockSpec(memory_space=pltpu.SMEM),
    )(logits, targets.reshape(_B, 1).astype(jnp.int32), tcodes)
    return out[0, 0]


# speedup vs baseline: 3.5022x; 1.0810x over previous
import functools

import jax
import jax.numpy as jnp
from jax import lax
from jax.experimental import pallas as pl
from jax.experimental.pallas import tpu as pltpu
from jax.experimental.pallas import tpu_sc as plsc

_B, _N = 64, 32768
_K1 = int(0.01 * (_N - 1)) + 1
_NVEC = _N // 16
_CAP = 32000
_MASK31 = 0x7FFFFFFF
_TAU = 0x40000000


def _sc_select_body(logits_hbm, out_hbm, row0_v, row1_v, cand0_v, cand1_v,
                    ebuf_v, sem):
    wid = lax.axis_index("s") * 2 + lax.axis_index("c")
    lane = lax.broadcasted_iota(jnp.int32, (16,), 0)
    r0 = wid * 2
    keys_hbm = logits_hbm.bitcast(jnp.int32)
    pltpu.sync_copy(keys_hbm.at[r0], row0_v)
    pltpu.sync_copy(keys_hbm.at[r0 + 1], row1_v)

    @plsc.parallel_loop(0, _NVEC, unroll=4,
                        carry=(jnp.int32(0), jnp.int32(0)))
    def offs(i, c):
        o0, o1 = c
        b0 = row0_v[pl.ds(i * 16, 16)]
        k0 = b0 ^ ((b0 >> 31) & _MASK31)
        m0 = k0 >= _TAU
        plsc.store_compressed(cand0_v.at[pl.ds(o0, 16)], k0,
                              mask=m0 & (o0 < _CAP))
        b1 = row1_v[pl.ds(i * 16, 16)]
        k1 = b1 ^ ((b1 >> 31) & _MASK31)
        m1 = k1 >= _TAU
        plsc.store_compressed(cand1_v.at[pl.ds(o1, 16)], k1,
                              mask=m1 & (o1 < _CAP))
        return (o0 + plsc.all_reduce_population_count(m0)[0],
                o1 + plsc.all_reduce_population_count(m1)[0])

    pad = jnp.full((16,), -2147483648, jnp.int32)

    for r_i, (row_v, cand_v, off) in enumerate(
            [(row0_v, cand0_v, offs[0]), (row1_v, cand1_v, offs[1])]):

        def fast_path(_, cand_v=cand_v, off=off):
            cand_v[pl.ds(off, 16)] = pad
            nv = (off + 15) // 16

            def outer(j, res):
                cnd = res + (jnp.int32(1) << (jnp.int32(29) - j))

                @plsc.parallel_loop(0, nv, unroll=4,
                                    carry=jnp.zeros((16,), jnp.int32))
                def acc(i, a, cnd=cnd, cand_v=cand_v):
                    key = cand_v[pl.ds(i * 16, 16)]
                    return a + jnp.where(key >= cnd, 1, 0).astype(jnp.int32)

                return jnp.where(jnp.sum(acc) >= _K1, cnd, res)

            return lax.fori_loop(0, 30, outer, jnp.int32(_TAU))

        def slow_path(_, row_v=row_v):
            def outer(j, res):
                cnd = res + (jnp.int32(1) << (jnp.int32(30) - j))

                @plsc.parallel_loop(0, _NVEC, unroll=4,
                                    carry=jnp.zeros((16,), jnp.int32))
                def acc(i, a, cnd=cnd, row_v=row_v):
                    b = row_v[pl.ds(i * 16, 16)]
                    key = b ^ ((b >> 31) & _MASK31)
                    return a + jnp.where(key >= cnd, 1, 0).astype(jnp.int32)

                return jnp.where(jnp.sum(acc) >= _K1, cnd, res)

            return lax.fori_loop(0, 31, outer, jnp.int32(-2147483648))

        ok = (off >= _K1) & (off <= _CAP)
        tcode = lax.cond(ok, fast_path, slow_path, jnp.int32(0))
        ebuf_v[pl.ds(0, 16)] = jnp.full((16,), tcode)
        pltpu.sync_copy(ebuf_v, out_hbm.at[r0 + r_i])


def _tc_finish_body(logits_ref, tgt_ref, tcode_ref, out_ref):
    x = logits_ref[...]
    b = lax.bitcast_convert_type(x, jnp.int32)
    key = b ^ ((b >> 31) & _MASK31)
    tcode = tcode_ref[:, 0:1]

    m = jnp.max(x, axis=1, keepdims=True)
    gt = key > tcode
    c = jnp.sum(gt.astype(jnp.int32), axis=1, keepdims=True)
    tsum = jnp.sum(jnp.where(gt, jnp.exp(10.0 * (x - m)), 0.0), axis=1,
                   keepdims=True)

    tb = tcode ^ ((tcode >> 31) & _MASK31)
    t = lax.bitcast_convert_type(tb, jnp.float32)
    cols = lax.broadcasted_iota(jnp.int32, (_B, _N), 1)
    pos = jnp.sum(jnp.where(cols == tgt_ref[...], x, 0.0), axis=1,
                  keepdims=True)

    et = jnp.exp(10.0 * (t - m))
    ep = jnp.exp(10.0 * (pos - m))
    s = tsum + (_K1 - c).astype(jnp.float32) * et + jnp.where(
        pos < t, ep - et, 0.0)
    loss = jnp.log(s) + 10.0 * m - 10.0 * pos
    out_ref[0, 0] = jnp.sum(loss) / _B


@jax.jit
def kernel(logits, targets):
    mesh = plsc.VectorSubcoreMesh(core_axis_name="c", subcore_axis_name="s")
    sc = functools.partial(
        pl.kernel,
        mesh=mesh,
        compiler_params=pltpu.CompilerParams(needs_layout_passes=False),
        out_type=jax.ShapeDtypeStruct((_B, 16), jnp.int32),
        scratch_types=[
            pltpu.VMEM((_N,), jnp.int32),
            pltpu.VMEM((_N,), jnp.int32),
            pltpu.VMEM((_CAP + 16,), jnp.int32),
            pltpu.VMEM((_CAP + 16,), jnp.int32),
            pltpu.VMEM((16,), jnp.int32),
            pltpu.SemaphoreType.DMA,
        ],
    )(_sc_select_body)
    tcodes = sc(logits)

    out = pl.pallas_call(
        _tc_finish_body,
        out_shape=jax.ShapeDtypeStruct((1, 1), jnp.float32),
        in_specs=[
            pl.BlockSpec(memory_space=pltpu.VMEM),
            pl.BlockSpec(memory_space=pltpu.VMEM),
            pl.BlockSpec(memory_space=pltpu.VMEM),
        ],
        out_specs=pl.BlockSpec(memory_space=pltpu.SMEM),
    )(logits, targets.reshape(_B, 1).astype(jnp.int32), tcodes)
    return out[0, 0]


# speedup vs baseline: 3.7111x; 1.0596x over previous
import functools

import jax
import jax.numpy as jnp
from jax import lax
from jax.experimental import pallas as pl
from jax.experimental.pallas import tpu as pltpu
from jax.experimental.pallas import tpu_sc as plsc

_B, _N = 64, 32768
_K1 = int(0.01 * (_N - 1)) + 1
_NVEC = _N // 16
_CAP = 32000
_MASK31 = 0x7FFFFFFF
_TAU = 0x40000000


def _sc_select_body(logits_hbm, out_hbm, row0_v, row1_v, cand0_v, cand1_v,
                    ebuf_v, sem):
    wid = lax.axis_index("s") * 2 + lax.axis_index("c")
    r0 = wid * 2
    keys_hbm = logits_hbm.bitcast(jnp.int32)
    pltpu.sync_copy(keys_hbm.at[r0], row0_v)
    h1 = pltpu.async_copy(keys_hbm.at[r0 + 1], row1_v, sem)

    def screen(row_v, cand_v):
        @plsc.parallel_loop(0, _NVEC, unroll=8, carry=jnp.int32(0))
        def off(i, o):
            b = row_v[pl.ds(i * 16, 16)]
            k = b ^ ((b >> 31) & _MASK31)
            m = k >= _TAU
            plsc.store_compressed(cand_v.at[pl.ds(o, 16)], k,
                                  mask=m & (o < _CAP))
            return o + plsc.all_reduce_population_count(m)[0]

        return off

    off0 = screen(row0_v, cand0_v)
    h1.wait()
    off1 = screen(row1_v, cand1_v)

    pad = jnp.full((16,), -2147483648, jnp.int32)

    for r_i, (row_v, cand_v, off) in enumerate(
            [(row0_v, cand0_v, off0), (row1_v, cand1_v, off1)]):

        def fast_path(_, cand_v=cand_v, off=off):
            cand_v[pl.ds(off, 16)] = pad
            nv = (off + 15) // 16

            def outer(j, res):
                cnd = res + (jnp.int32(1) << (jnp.int32(29) - j))

                @plsc.parallel_loop(0, nv, unroll=4,
                                    carry=jnp.zeros((16,), jnp.int32))
                def acc(i, a, cnd=cnd, cand_v=cand_v):
                    key = cand_v[pl.ds(i * 16, 16)]
                    return a + jnp.where(key >= cnd, 1, 0).astype(jnp.int32)

                return jnp.where(jnp.sum(acc) >= _K1, cnd, res)

            return lax.fori_loop(0, 30, outer, jnp.int32(_TAU))

        def slow_path(_, row_v=row_v):
            def outer(j, res):
                cnd = res + (jnp.int32(1) << (jnp.int32(30) - j))

                @plsc.parallel_loop(0, _NVEC, unroll=4,
                                    carry=jnp.zeros((16,), jnp.int32))
                def acc(i, a, cnd=cnd, row_v=row_v):
                    b = row_v[pl.ds(i * 16, 16)]
                    key = b ^ ((b >> 31) & _MASK31)
                    return a + jnp.where(key >= cnd, 1, 0).astype(jnp.int32)

                return jnp.where(jnp.sum(acc) >= _K1, cnd, res)

            return lax.fori_loop(0, 31, outer, jnp.int32(-2147483648))

        ok = (off >= _K1) & (off <= _CAP)
        tcode = lax.cond(ok, fast_path, slow_path, jnp.int32(0))
        ebuf_v[pl.ds(0, 16)] = jnp.full((16,), tcode)
        pltpu.sync_copy(ebuf_v, out_hbm.at[r0 + r_i])


def _tc_stats_body(logits_ref, tgt_ref, m_ref, pos_ref):
    x = logits_ref[...]
    m_ref[...] = jnp.max(x, axis=1, keepdims=True)
    cols = lax.broadcasted_iota(jnp.int32, (_B, _N), 1)
    pos_ref[...] = jnp.sum(jnp.where(cols == tgt_ref[...], x, 0.0), axis=1,
                           keepdims=True)


def _tc_finish_body(logits_ref, tcode_ref, m_ref, pos_ref, out_ref):
    x = logits_ref[...]
    b = lax.bitcast_convert_type(x, jnp.int32)
    key = b ^ ((b >> 31) & _MASK31)
    tcode = tcode_ref[:, 0:1]
    m = m_ref[...]
    pos = pos_ref[...]

    gt = key > tcode
    c = jnp.sum(gt.astype(jnp.int32), axis=1, keepdims=True)
    tsum = jnp.sum(jnp.where(gt, jnp.exp(10.0 * (x - m)), 0.0), axis=1,
                   keepdims=True)

    tb = tcode ^ ((tcode >> 31) & _MASK31)
    t = lax.bitcast_convert_type(tb, jnp.float32)
    et = jnp.exp(10.0 * (t - m))
    ep = jnp.exp(10.0 * (pos - m))
    s = tsum + (_K1 - c).astype(jnp.float32) * et + jnp.where(
        pos < t, ep - et, 0.0)
    loss = jnp.log(s) + 10.0 * m - 10.0 * pos
    out_ref[0, 0] = jnp.sum(loss) / _B


@jax.jit
def kernel(logits, targets):
    mesh = plsc.VectorSubcoreMesh(core_axis_name="c", subcore_axis_name="s")
    sc = functools.partial(
        pl.kernel,
        mesh=mesh,
        compiler_params=pltpu.CompilerParams(needs_layout_passes=False),
        out_type=jax.ShapeDtypeStruct((_B, 16), jnp.int32),
        scratch_types=[
            pltpu.VMEM((_N,), jnp.int32),
            pltpu.VMEM((_N,), jnp.int32),
            pltpu.VMEM((_CAP + 16,), jnp.int32),
            pltpu.VMEM((_CAP + 16,), jnp.int32),
            pltpu.VMEM((16,), jnp.int32),
            pltpu.SemaphoreType.DMA,
        ],
    )(_sc_select_body)
    tcodes = sc(logits)

    m, pos = pl.pallas_call(
        _tc_stats_body,
        out_shape=[
            jax.ShapeDtypeStruct((_B, 1), jnp.float32),
            jax.ShapeDtypeStruct((_B, 1), jnp.float32),
        ],
        in_specs=[
            pl.BlockSpec(memory_space=pltpu.VMEM),
            pl.BlockSpec(memory_space=pltpu.VMEM),
        ],
        out_specs=[
            pl.BlockSpec(memory_space=pltpu.VMEM),
            pl.BlockSpec(memory_space=pltpu.VMEM),
        ],
    )(logits, targets.reshape(_B, 1).astype(jnp.int32))

    out = pl.pallas_call(
        _tc_finish_body,
        out_shape=jax.ShapeDtypeStruct((1, 1), jnp.float32),
        in_specs=[
            pl.BlockSpec(memory_space=pltpu.VMEM),
            pl.BlockSpec(memory_space=pltpu.VMEM),
            pl.BlockSpec(memory_space=pltpu.VMEM),
            pl.BlockSpec(memory_space=pltpu.VMEM),
        ],
        out_specs=pl.BlockSpec(memory_space=pltpu.SMEM),
    )(logits, tcodes, m, pos)
    return out[0, 0]
